# baseline (speedup 1.0000x reference)
import jax
import jax.numpy as jnp
from jax.experimental import pallas as pl
from jax.experimental.pallas import tpu as pltpu

T, H, E, F, TOP_K = 32, 2048, 16, 1408, 2
HB = 512
NH = H // HB
HBO = 1024
ND = H // HBO
NS = NH + ND


def _routing_scale(logits, e):
    iota = jax.lax.broadcasted_iota(jnp.int32, (T, E), 1)
    m1 = jnp.max(logits, axis=1, keepdims=True)
    idx1 = jnp.min(jnp.where(logits >= m1, iota, E), axis=1, keepdims=True)
    masked = jnp.where(iota == idx1, -jnp.inf, logits)
    m2 = jnp.max(masked, axis=1, keepdims=True)
    idx2 = jnp.min(jnp.where(masked >= m2, iota, E), axis=1, keepdims=True)
    w1 = 1.0 / (1.0 + jnp.exp(m2 - m1))
    w2 = 1.0 - w1
    return jnp.where(idx1 == e, w1, 0.0) + jnp.where(idx2 == e, w2, 0.0)


def _moe_kernel(x_ref, logits_ref, wg_ref, wu_ref, wd_ref, out_ref,
                g_ref, u_ref, act_ref):
    e = pl.program_id(0)
    s = pl.program_id(1)

    @pl.when(s < NH)
    def _phase_a():
        x = x_ref[...].astype(jnp.bfloat16)
        g = jnp.dot(x, wg_ref[0].astype(jnp.bfloat16),
                    preferred_element_type=jnp.float32)
        u = jnp.dot(x, wu_ref[0].astype(jnp.bfloat16),
                    preferred_element_type=jnp.float32)

        @pl.when(s == 0)
        def _():
            g_ref[...] = g
            u_ref[...] = u

        @pl.when(s > 0)
        def _():
            g_ref[...] += g
            u_ref[...] += u

    @pl.when(s == NH)
    def _make_act():
        g = g_ref[...]
        u = u_ref[...]
        scale = _routing_scale(logits_ref[...], e)
        act_ref[...] = scale * ((g * jax.lax.logistic(g)) * u)

    @pl.when(s >= NH)
    def _phase_b():
        partial = jnp.dot(act_ref[...].astype(jnp.bfloat16),
                          wd_ref[0].astype(jnp.bfloat16),
                          preferred_element_type=jnp.float32)

        @pl.when(e == 0)
        def _():
            out_ref[...] = partial

        @pl.when(e > 0)
        def _():
            out_ref[...] += partial


def kernel(hidden_states, router_logits, W_gate, W_up, W_down):
    clamp_a = lambda s: jnp.minimum(s, NH - 1)
    clamp_b = lambda s: jnp.clip(s - NH, 0, ND - 1)
    return pl.pallas_call(
        _moe_kernel,
        grid=(E, NS),
        in_specs=[
            pl.BlockSpec((T, HB), lambda e, s: (0, clamp_a(s))),
            pl.BlockSpec((T, E), lambda e, s: (0, 0)),
            pl.BlockSpec((1, HB, F), lambda e, s: (e, clamp_a(s), 0)),
            pl.BlockSpec((1, HB, F), lambda e, s: (e, clamp_a(s), 0)),
            pl.BlockSpec((1, F, HBO), lambda e, s: (e, 0, clamp_b(s))),
        ],
        out_specs=pl.BlockSpec((T, HBO), lambda e, s: (0, clamp_b(s))),
        out_shape=jax.ShapeDtypeStruct((T, H), jnp.float32),
        scratch_shapes=[
            pltpu.VMEM((T, F), jnp.float32),
            pltpu.VMEM((T, F), jnp.float32),
            pltpu.VMEM((T, F), jnp.float32),
        ],
        compiler_params=pltpu.CompilerParams(
            dimension_semantics=("arbitrary", "arbitrary"),
            vmem_limit_bytes=64 * 1024 * 1024,
        ),
    )(hidden_states, router_logits, W_gate, W_up, W_down)


# speedup vs baseline: 1.1836x; 1.1836x over previous
import jax
import jax.numpy as jnp
from jax.experimental import pallas as pl
from jax.experimental.pallas import tpu as pltpu

T, H, E, F, TOP_K = 32, 2048, 16, 1408, 2
HH = H // 2
FH = F // 2


def _routing_scale(logits, e):
    iota = jax.lax.broadcasted_iota(jnp.int32, (T, E), 1)
    m1 = jnp.max(logits, axis=1, keepdims=True)
    idx1 = jnp.min(jnp.where(logits >= m1, iota, E), axis=1, keepdims=True)
    masked = jnp.where(iota == idx1, -jnp.inf, logits)
    m2 = jnp.max(masked, axis=1, keepdims=True)
    idx2 = jnp.min(jnp.where(masked >= m2, iota, E), axis=1, keepdims=True)
    w1 = 1.0 / (1.0 + jnp.exp(m2 - m1))
    w2 = 1.0 - w1
    return jnp.where(idx1 == e, w1, 0.0) + jnp.where(idx2 == e, w2, 0.0)


def _moe_kernel(x_ref, logits_ref, wg_hbm, wu_hbm, wd_hbm, out_ref,
                a_buf, d_buf, a_sem, d_sem):
    def issue_a(e, j):
        w = wg_hbm if j < 2 else wu_hbm
        h0 = (j % 2) * HH
        pltpu.make_async_copy(
            w.at[e, pl.ds(h0, HH), :], a_buf.at[j], a_sem.at[j]).start()

    def issue_d(e, j):
        pltpu.make_async_copy(
            wd_hbm.at[e, pl.ds(j * FH, FH), :], d_buf.at[j], d_sem.at[j]).start()

    for j in range(4):
        issue_a(0, j)
    for j in range(2):
        issue_d(0, j)

    x = x_ref[...]
    x0 = x[:, :HH]
    x1 = x[:, HH:]
    logits = logits_ref[...]

    out_ref[...] = jnp.zeros((T, H), dtype=jnp.float32)

    def body(e, carry):
        def consume_a(j, xh):
            pltpu.make_async_copy(a_buf.at[j], a_buf.at[j], a_sem.at[j]).wait()
            r = jnp.dot(xh, a_buf[j], preferred_element_type=jnp.float32)

            @pl.when(e + 1 < E)
            def _():
                issue_a(e + 1, j)
            return r

        g = consume_a(0, x0)
        g += consume_a(1, x1)
        u = consume_a(2, x0)
        u += consume_a(3, x1)

        scale = _routing_scale(logits, e)
        act = scale * ((g * jax.lax.logistic(g)) * u)

        def consume_d(j):
            pltpu.make_async_copy(d_buf.at[j], d_buf.at[j], d_sem.at[j]).wait()
            out_ref[...] += jnp.dot(act[:, j * FH:(j + 1) * FH], d_buf[j],
                                    preferred_element_type=jnp.float32)

            @pl.when(e + 1 < E)
            def _():
                issue_d(e + 1, j)

        consume_d(0)
        consume_d(1)
        return carry

    jax.lax.fori_loop(0, E, body, 0)


def kernel(hidden_states, router_logits, W_gate, W_up, W_down):
    return pl.pallas_call(
        _moe_kernel,
        in_specs=[
            pl.BlockSpec(memory_space=pltpu.VMEM),
            pl.BlockSpec(memory_space=pltpu.VMEM),
            pl.BlockSpec(memory_space=pl.ANY),
            pl.BlockSpec(memory_space=pl.ANY),
            pl.BlockSpec(memory_space=pl.ANY),
        ],
        out_specs=pl.BlockSpec(memory_space=pltpu.VMEM),
        out_shape=jax.ShapeDtypeStruct((T, H), jnp.float32),
        scratch_shapes=[
            pltpu.VMEM((4, HH, F), jnp.float32),
            pltpu.VMEM((2, FH, H), jnp.float32),
            pltpu.SemaphoreType.DMA((4,)),
            pltpu.SemaphoreType.DMA((2,)),
        ],
        compiler_params=pltpu.CompilerParams(
            vmem_limit_bytes=60 * 1024 * 1024,
        ),
    )(hidden_states, router_logits, W_gate, W_up, W_down)


# speedup vs baseline: 1.1900x; 1.0055x over previous
import jax
import jax.numpy as jnp
from jax.experimental import pallas as pl
from jax.experimental.pallas import tpu as pltpu

T, H, E, F, TOP_K = 32, 2048, 16, 1408, 2
HH = H // 2
FH = F // 2


def _routing_scale(logits, e):
    iota = jax.lax.broadcasted_iota(jnp.int32, (T, E), 1)
    m1 = jnp.max(logits, axis=1, keepdims=True)
    idx1 = jnp.min(jnp.where(logits >= m1, iota, E), axis=1, keepdims=True)
    masked = jnp.where(iota == idx1, -jnp.inf, logits)
    m2 = jnp.max(masked, axis=1, keepdims=True)
    idx2 = jnp.min(jnp.where(masked >= m2, iota, E), axis=1, keepdims=True)
    w1 = 1.0 / (1.0 + jnp.exp(m2 - m1))
    w2 = 1.0 - w1
    return jnp.where(idx1 == e, w1, 0.0) + jnp.where(idx2 == e, w2, 0.0)


def _moe_kernel(x_ref, logits_ref, wg_hbm, wu_hbm, wd_hbm, out_ref,
                a_buf, d_buf, a_sem, d_sem):
    def issue_a(e, j, slot):
        w = wg_hbm if j < 2 else wu_hbm
        h0 = (j % 2) * HH
        pltpu.make_async_copy(
            w.at[e, pl.ds(h0, HH), :], a_buf.at[slot], a_sem.at[slot]).start()

    def issue_d(e, j, slot):
        pltpu.make_async_copy(
            wd_hbm.at[e, pl.ds(j * FH, FH), :],
            d_buf.at[slot], d_sem.at[slot]).start()

    for c in range(6):
        issue_a(c // 4, c % 4, c)
    for c in range(3):
        issue_d(c // 2, c % 2, c)

    x = x_ref[...]
    x0 = x[:, :HH]
    x1 = x[:, HH:]
    logits = logits_ref[...]

    out_ref[...] = jnp.zeros((T, H), dtype=jnp.float32)

    def body(e, carry):
        def consume_a(j, xh):
            sa = jax.lax.rem(4 * e + j, 6)
            pltpu.make_async_copy(
                a_buf.at[sa], a_buf.at[sa], a_sem.at[sa]).wait()
            r = jnp.dot(xh, a_buf[sa], preferred_element_type=jnp.float32)
            e_i = e + 1 if j < 2 else e + 2
            j_i = (j + 2) % 4

            @pl.when(e_i < E)
            def _():
                issue_a(e_i, j_i, sa)
            return r

        g = consume_a(0, x0)
        g += consume_a(1, x1)
        u = consume_a(2, x0)
        u += consume_a(3, x1)

        scale = _routing_scale(logits, e)
        act = scale * ((g * jax.lax.logistic(g)) * u)

        def consume_d(j):
            sd = jax.lax.rem(2 * e + j, 3)
            pltpu.make_async_copy(
                d_buf.at[sd], d_buf.at[sd], d_sem.at[sd]).wait()
            out_ref[...] += jnp.dot(act[:, j * FH:(j + 1) * FH], d_buf[sd],
                                    preferred_element_type=jnp.float32)
            e_i = e + 1 if j == 0 else e + 2
            j_i = 1 - j

            @pl.when(e_i < E)
            def _():
                issue_d(e_i, j_i, sd)

        consume_d(0)
        consume_d(1)
        return carry

    jax.lax.fori_loop(0, E, body, 0)


def kernel(hidden_states, router_logits, W_gate, W_up, W_down):
    return pl.pallas_call(
        _moe_kernel,
        in_specs=[
            pl.BlockSpec(memory_space=pltpu.VMEM),
            pl.BlockSpec(memory_space=pltpu.VMEM),
            pl.BlockSpec(memory_space=pl.ANY),
            pl.BlockSpec(memory_space=pl.ANY),
            pl.BlockSpec(memory_space=pl.ANY),
        ],
        out_specs=pl.BlockSpec(memory_space=pltpu.VMEM),
        out_shape=jax.ShapeDtypeStruct((T, H), jnp.float32),
        scratch_shapes=[
            pltpu.VMEM((6, HH, F), jnp.float32),
            pltpu.VMEM((3, FH, H), jnp.float32),
            pltpu.SemaphoreType.DMA((6,)),
            pltpu.SemaphoreType.DMA((3,)),
        ],
        compiler_params=pltpu.CompilerParams(
            vmem_limit_bytes=60 * 1024 * 1024,
        ),
    )(hidden_states, router_logits, W_gate, W_up, W_down)
